# baseline (speedup 1.0000x reference)
import functools

import jax
import jax.numpy as jnp
from jax import lax
from jax.experimental import pallas as pl
from jax.experimental.pallas import tpu as pltpu
from jax.experimental.pallas import tpu_sc as plsc

D = 64
NC, NS = 2, 16
NW = NC * NS
CHUNK = 128
L = 16
NJ = 50
NI = 16384


def _cast_block(rows_ref, stg_ref):
    iota2 = 2 * lax.iota(jnp.int32, L)

    def d_body(d, _):
        dv = jnp.full((L,), d, jnp.int32)
        for g in range(CHUNK // (2 * L)):
            ile = g * 2 * L + iota2
            e = plsc.load_gather(rows_ref, [ile, dv])
            o = plsc.load_gather(rows_ref, [ile + 1, dv])
            pk = plsc.pack(e, o, format=plsc.PackFormat.INTERLEAVED)
            stg_ref[d, pl.ds(2 * g * L, 2 * L)] = pk
        return 0

    lax.fori_loop(0, D, d_body, 0, unroll=4)


def _emb_body(n_chunk, x_hbm, w_hbm, out_hbm, idx_v, r0, r1, s0, s1,
              sem_i, g0, g1, w0, w1):
    wid = lax.axis_index("s") * NC + lax.axis_index("c")
    base = wid * n_chunk
    pltpu.async_copy(x_hbm.at[wid], idx_v, sem_i).wait()

    rows = (r0, r1)
    stgs = (s0, s1)
    gs = (g0, g1)
    ws = (w0, w1)

    def fire_gather(c, b):
        pltpu.async_copy(w_hbm.at[idx_v.at[c]], rows[b], gs[b])

    def wait_gather(c, b):
        pltpu.make_async_copy(w_hbm.at[idx_v.at[c]], rows[b], gs[b]).wait()

    def fire_write(c, b):
        blk = base + c
        j = blk // (NI // CHUNK)
        i0 = (blk % (NI // CHUNK)) * CHUNK
        pltpu.async_copy(stgs[b], out_hbm.at[j, :, pl.ds(i0, CHUNK)], ws[b])

    def drain_write(b):
        pltpu.make_async_copy(stgs[b], out_hbm.at[0, :, pl.ds(0, CHUNK)],
                              ws[b]).wait()

    fire_gather(0, 0)
    fire_gather(1, 1)

    for b in range(2):
        wait_gather(b, b)
        _cast_block(rows[b], stgs[b])
        fire_write(b, b)
        fire_gather(b + 2, b)

    half = n_chunk // 2

    def main_body(k, _):
        c = 2 * k
        for b in range(2):
            wait_gather(c + b, b)
            drain_write(b)
            _cast_block(rows[b], stgs[b])
            fire_write(c + b, b)
            fire_gather(c + b + 2, b)
        return 0

    lax.fori_loop(1, half - 1, main_body, 0)

    for b in range(2):
        c = n_chunk - 2 + b
        wait_gather(c, b)
        drain_write(b)
        _cast_block(rows[b], stgs[b])
        fire_write(c, b)

    drain_write(0)
    drain_write(1)


def _build(n_chunk):
    mesh = plsc.VectorSubcoreMesh(core_axis_name="c", subcore_axis_name="s")
    return pl.kernel(
        functools.partial(_emb_body, n_chunk),
        out_type=jax.ShapeDtypeStruct((NJ, D, NI), jnp.bfloat16),
        mesh=mesh,
        scratch_types=[
            pltpu.VMEM((n_chunk, CHUNK), jnp.int32),
            pltpu.VMEM((CHUNK, 128), jnp.float32),
            pltpu.VMEM((CHUNK, 128), jnp.float32),
            pltpu.VMEM((D, CHUNK), jnp.bfloat16),
            pltpu.VMEM((D, CHUNK), jnp.bfloat16),
            pltpu.SemaphoreType.DMA,
            pltpu.SemaphoreType.DMA,
            pltpu.SemaphoreType.DMA,
            pltpu.SemaphoreType.DMA,
            pltpu.SemaphoreType.DMA,
        ],
        compiler_params=pltpu.CompilerParams(
            use_tc_tiling_on_sc=False, needs_layout_passes=False),
    )


BN = 2048


def _tr_body(in_ref, out_ref):
    a = in_ref[...]
    out_ref[:, 0:D] = a.T


def _transpose_table(wt):
    n = wt.shape[1]
    return pl.pallas_call(
        _tr_body,
        grid=(pl.cdiv(n, BN),),
        in_specs=[pl.BlockSpec((D, BN), lambda i: (0, i))],
        out_specs=pl.BlockSpec((BN, 128), lambda i: (i, 0)),
        out_shape=jax.ShapeDtypeStruct((n, 128), jnp.float32),
    )(wt)


def kernel(x, weight):
    n_total = x.size
    n_chunk = n_total // (NW * CHUNK)
    xw = x.T.reshape(NW, n_chunk, CHUNK).astype(jnp.int32)
    table = _transpose_table(weight.T)
    o3 = _build(n_chunk)(xw, table)
    return o3.transpose(2, 0, 1)


# speedup vs baseline: 1.3326x; 1.3326x over previous
import functools

import jax
import jax.numpy as jnp
from jax import lax
from jax.experimental import pallas as pl
from jax.experimental.pallas import tpu as pltpu
from jax.experimental.pallas import tpu_sc as plsc

D = 64
NC, NS = 2, 16
NW = NC * NS
CHUNK = 128
L = 16
NJ = 50
NI = 16384


def _cast_block(rows_ref, stg_ref):
    iota = lax.iota(jnp.int32, L)

    def d_body(d, _):
        dk = (d + iota) & (D - 1)
        for g in range(CHUNK // (2 * L)):
            re = g * 2 * L + 2 * iota
            e = plsc.load_gather(rows_ref, [re, dk])
            o = plsc.load_gather(rows_ref, [re + 1, dk])
            pk = plsc.pack(e, o, format=plsc.PackFormat.INTERLEAVED)
            w = plsc.bitcast(pk, jnp.int32)
            plsc.store_scatter(stg_ref, [dk, g * L + iota], w)
        return 0

    lax.fori_loop(0, D, d_body, 0, unroll=4)


def _emb_body(n_chunk, x_hbm, w_hbm, out_hbm, idx_v, r0, r1, s0, s1,
              sem_i, g0, g1, w0, w1):
    wid = lax.axis_index("s") * NC + lax.axis_index("c")
    base = wid * n_chunk
    pltpu.async_copy(x_hbm.at[wid], idx_v, sem_i).wait()

    rows = (r0, r1)
    stgs = (s0, s1)
    gs = (g0, g1)
    ws = (w0, w1)

    def fire_gather(c, b):
        pltpu.async_copy(w_hbm.at[idx_v.at[c]], rows[b], gs[b])

    def wait_gather(c, b):
        pltpu.make_async_copy(w_hbm.at[idx_v.at[c]], rows[b], gs[b]).wait()

    def fire_write(c, b):
        blk = base + c
        j = blk // (NI // CHUNK)
        w0_ = (blk % (NI // CHUNK)) * (CHUNK // 2)
        pltpu.async_copy(stgs[b], out_hbm.at[j, :, pl.ds(w0_, CHUNK // 2)],
                         ws[b])

    def drain_write(b):
        pltpu.make_async_copy(stgs[b], out_hbm.at[0, :, pl.ds(0, CHUNK // 2)],
                              ws[b]).wait()

    fire_gather(0, 0)
    fire_gather(1, 1)

    for b in range(2):
        wait_gather(b, b)
        _cast_block(rows[b], stgs[b])
        fire_write(b, b)
        fire_gather(b + 2, b)

    half = n_chunk // 2

    def main_body(k, _):
        c = 2 * k
        for b in range(2):
            wait_gather(c + b, b)
            drain_write(b)
            _cast_block(rows[b], stgs[b])
            fire_write(c + b, b)
            fire_gather(c + b + 2, b)
        return 0

    lax.fori_loop(1, half - 1, main_body, 0)

    for b in range(2):
        c = n_chunk - 2 + b
        wait_gather(c, b)
        drain_write(b)
        _cast_block(rows[b], stgs[b])
        fire_write(c, b)

    drain_write(0)
    drain_write(1)


def _build(n_chunk):
    mesh = plsc.VectorSubcoreMesh(core_axis_name="c", subcore_axis_name="s")
    return pl.kernel(
        functools.partial(_emb_body, n_chunk),
        out_type=jax.ShapeDtypeStruct((NJ, D, NI // 2), jnp.int32),
        mesh=mesh,
        scratch_types=[
            pltpu.VMEM((n_chunk, CHUNK), jnp.int32),
            pltpu.VMEM((CHUNK, 128), jnp.float32),
            pltpu.VMEM((CHUNK, 128), jnp.float32),
            pltpu.VMEM((D, CHUNK // 2), jnp.int32),
            pltpu.VMEM((D, CHUNK // 2), jnp.int32),
            pltpu.SemaphoreType.DMA,
            pltpu.SemaphoreType.DMA,
            pltpu.SemaphoreType.DMA,
            pltpu.SemaphoreType.DMA,
            pltpu.SemaphoreType.DMA,
        ],
        compiler_params=pltpu.CompilerParams(
            use_tc_tiling_on_sc=False, needs_layout_passes=False),
    )


BN = 2048


def _tr_body(in_ref, out_ref):
    a = in_ref[...]
    out_ref[:, 0:D] = a.T


def _transpose_table(wt):
    n = wt.shape[1]
    return pl.pallas_call(
        _tr_body,
        grid=(pl.cdiv(n, BN),),
        in_specs=[pl.BlockSpec((D, BN), lambda i: (0, i))],
        out_specs=pl.BlockSpec((BN, 128), lambda i: (i, 0)),
        out_shape=jax.ShapeDtypeStruct((n, 128), jnp.float32),
    )(wt)


def kernel(x, weight):
    n_total = x.size
    n_chunk = n_total // (NW * CHUNK)
    xw = x.T.reshape(NW, n_chunk, CHUNK).astype(jnp.int32)
    table = _transpose_table(weight.T)
    ou = _build(n_chunk)(xw, table)
    o3 = lax.bitcast_convert_type(ou, jnp.bfloat16).reshape(NJ, D, NI)
    return o3.transpose(2, 0, 1)


# speedup vs baseline: 1.9490x; 1.4625x over previous
import functools

import jax
import jax.numpy as jnp
from jax import lax
from jax.experimental import pallas as pl
from jax.experimental.pallas import tpu as pltpu
from jax.experimental.pallas import tpu_sc as plsc

D = 64
NC, NS = 2, 16
NW = NC * NS
CHUNK = 128
L = 16
NJ = 50
NI = 16384


def _cast_block(rows_ref, stg_ref):
    iota = lax.iota(jnp.int32, L)

    def rr_body(rr, _):
        rk = (rr + iota) & (D // 2 - 1)
        dk = 2 * rk
        for g in range(CHUNK // L):
            cl = g * L + iota
            e = plsc.load_gather(rows_ref, [cl, dk])
            o = plsc.load_gather(rows_ref, [cl, dk + 1])
            pk = plsc.pack(e, o, format=plsc.PackFormat.INTERLEAVED)
            w = plsc.bitcast(pk, jnp.int32)
            plsc.store_scatter(stg_ref, [rk, cl], w)
        return 0

    lax.fori_loop(0, D // 2, rr_body, 0, unroll=4)


def _emb_body(n_chunk, x_hbm, w_hbm, out_hbm, idx_v, r0, r1, s0, s1,
              sem_i, g0, g1, w0, w1):
    wid = lax.axis_index("s") * NC + lax.axis_index("c")
    base = wid * n_chunk
    pltpu.async_copy(x_hbm.at[wid], idx_v, sem_i).wait()

    rows = (r0, r1)
    stgs = (s0, s1)
    gs = (g0, g1)
    ws = (w0, w1)

    def fire_gather(c, b):
        pltpu.async_copy(w_hbm.at[idx_v.at[c]], rows[b], gs[b])

    def wait_gather(c, b):
        pltpu.make_async_copy(w_hbm.at[idx_v.at[c]], rows[b], gs[b]).wait()

    def fire_write(c, b):
        blk = base + c
        j = blk // (NI // CHUNK)
        tc = blk % (NI // CHUNK)
        pltpu.async_copy(stgs[b], out_hbm.at[j, tc], ws[b])

    def drain_write(b):
        pltpu.make_async_copy(stgs[b], out_hbm.at[0, 0], ws[b]).wait()

    fire_gather(0, 0)
    fire_gather(1, 1)

    for b in range(2):
        wait_gather(b, b)
        _cast_block(rows[b], stgs[b])
        fire_write(b, b)
        fire_gather(b + 2, b)

    half = n_chunk // 2

    def main_body(k, _):
        c = 2 * k
        for b in range(2):
            wait_gather(c + b, b)
            drain_write(b)
            _cast_block(rows[b], stgs[b])
            fire_write(c + b, b)
            fire_gather(c + b + 2, b)
        return 0

    lax.fori_loop(1, half - 1, main_body, 0)

    for b in range(2):
        c = n_chunk - 2 + b
        wait_gather(c, b)
        drain_write(b)
        _cast_block(rows[b], stgs[b])
        fire_write(c, b)

    drain_write(0)
    drain_write(1)


def _build(n_chunk):
    mesh = plsc.VectorSubcoreMesh(core_axis_name="c", subcore_axis_name="s")
    return pl.kernel(
        functools.partial(_emb_body, n_chunk),
        out_type=jax.ShapeDtypeStruct((NJ, NI // CHUNK, D // 2, CHUNK),
                                      jnp.int32),
        mesh=mesh,
        scratch_types=[
            pltpu.VMEM((n_chunk, CHUNK), jnp.int32),
            pltpu.VMEM((CHUNK, 128), jnp.float32),
            pltpu.VMEM((CHUNK, 128), jnp.float32),
            pltpu.VMEM((D // 2, CHUNK), jnp.int32),
            pltpu.VMEM((D // 2, CHUNK), jnp.int32),
            pltpu.SemaphoreType.DMA,
            pltpu.SemaphoreType.DMA,
            pltpu.SemaphoreType.DMA,
            pltpu.SemaphoreType.DMA,
            pltpu.SemaphoreType.DMA,
        ],
        compiler_params=pltpu.CompilerParams(
            use_tc_tiling_on_sc=False, needs_layout_passes=False),
    )


BN = 2048


def _tr_body(in_ref, out_ref):
    a = in_ref[...]
    eye = jnp.eye(D, dtype=jnp.float32)
    t = lax.dot_general(a, eye, (((0,), (0,)), ((), ())),
                        preferred_element_type=jnp.float32)
    out_ref[:, 0:D] = t


def _transpose_table(wt):
    n = wt.shape[1]
    return pl.pallas_call(
        _tr_body,
        grid=(pl.cdiv(n, BN),),
        in_specs=[pl.BlockSpec((D, BN), lambda i: (0, i))],
        out_specs=pl.BlockSpec((BN, 128), lambda i: (i, 0)),
        out_shape=jax.ShapeDtypeStruct((n, 128), jnp.float32),
    )(wt)


TPJ = 16


def _fmt_body(in_ref, out_ref):
    b = pltpu.bitcast(in_ref[...], jnp.bfloat16)
    for t in range(TPJ):
        out_ref[0, :, pl.ds(CHUNK * t, CHUNK)] = b[D * t:D * (t + 1), :]


def _format_out(ou2):
    rpb = TPJ * D // 2
    return pl.pallas_call(
        _fmt_body,
        grid=(NJ, NI // CHUNK // TPJ),
        in_specs=[pl.BlockSpec((rpb, 128),
                               lambda j, u: (j * (NI // CHUNK // TPJ) + u, 0))],
        out_specs=pl.BlockSpec((1, D, TPJ * CHUNK), lambda j, u: (j, 0, u)),
        out_shape=jax.ShapeDtypeStruct((NJ, D, NI), jnp.bfloat16),
    )(ou2)


def kernel(x, weight):
    n_total = x.size
    n_chunk = n_total // (NW * CHUNK)
    xw = x.T.reshape(NW, n_chunk, CHUNK).astype(jnp.int32)
    table = _transpose_table(weight.T)
    ou = _build(n_chunk)(xw, table)
    o3 = _format_out(ou.reshape(NJ * (NI // CHUNK) * (D // 2), CHUNK))
    return o3.transpose(2, 0, 1)


# speedup vs baseline: 2.3984x; 1.2306x over previous
import functools

import jax
import jax.numpy as jnp
from jax import lax
from jax.experimental import pallas as pl
from jax.experimental.pallas import tpu as pltpu
from jax.experimental.pallas import tpu_sc as plsc

D = 64
NC, NS = 2, 16
NW = NC * NS
CHUNK = 128
L = 16
NJ = 50
NI = 16384


def _cast_block(rows_ref, stg_ref):
    iota = lax.iota(jnp.int32, L)

    def rr_body(rr, _):
        rk = (rr + iota) & (D // 2 - 1)
        for g in range(CHUNK // L):
            cl = g * L + iota
            w = plsc.load_gather(rows_ref, [cl, rk])
            plsc.store_scatter(stg_ref, [rk, cl], w)
        return 0

    lax.fori_loop(0, D // 2, rr_body, 0, unroll=4)


def _emb_body(n_chunk, x_hbm, w_hbm, out_hbm, idx_v, r0, r1, s0, s1,
              sem_i, g0, g1, w0, w1):
    wid = lax.axis_index("s") * NC + lax.axis_index("c")
    base = wid * n_chunk
    pltpu.async_copy(x_hbm.at[wid], idx_v, sem_i).wait()

    rows = (r0, r1)
    stgs = (s0, s1)
    gs = (g0, g1)
    ws = (w0, w1)

    def fire_gather(c, b):
        pltpu.async_copy(w_hbm.at[idx_v.at[c]], rows[b], gs[b])

    def wait_gather(c, b):
        pltpu.make_async_copy(w_hbm.at[idx_v.at[c]], rows[b], gs[b]).wait()

    def fire_write(c, b):
        blk = base + c
        j = blk // (NI // CHUNK)
        tc = blk % (NI // CHUNK)
        pltpu.async_copy(stgs[b], out_hbm.at[j, tc], ws[b])

    def drain_write(b):
        pltpu.make_async_copy(stgs[b], out_hbm.at[0, 0], ws[b]).wait()

    fire_gather(0, 0)
    fire_gather(1, 1)

    for b in range(2):
        wait_gather(b, b)
        _cast_block(rows[b], stgs[b])
        fire_write(b, b)
        fire_gather(b + 2, b)

    half = n_chunk // 2

    def main_body(k, _):
        c = 2 * k
        for b in range(2):
            wait_gather(c + b, b)
            drain_write(b)
            _cast_block(rows[b], stgs[b])
            fire_write(c + b, b)
            fire_gather(c + b + 2, b)
        return 0

    lax.fori_loop(1, half - 1, main_body, 0)

    for b in range(2):
        c = n_chunk - 2 + b
        wait_gather(c, b)
        drain_write(b)
        _cast_block(rows[b], stgs[b])
        fire_write(c, b)

    drain_write(0)
    drain_write(1)


def _build(n_chunk):
    mesh = plsc.VectorSubcoreMesh(core_axis_name="c", subcore_axis_name="s")
    return pl.kernel(
        functools.partial(_emb_body, n_chunk),
        out_type=jax.ShapeDtypeStruct((NJ, NI // CHUNK, D // 2, CHUNK),
                                      jnp.int32),
        mesh=mesh,
        scratch_types=[
            pltpu.VMEM((n_chunk, CHUNK), jnp.int32),
            pltpu.VMEM((CHUNK, 128), jnp.int32),
            pltpu.VMEM((CHUNK, 128), jnp.int32),
            pltpu.VMEM((D // 2, CHUNK), jnp.int32),
            pltpu.VMEM((D // 2, CHUNK), jnp.int32),
            pltpu.SemaphoreType.DMA,
            pltpu.SemaphoreType.DMA,
            pltpu.SemaphoreType.DMA,
            pltpu.SemaphoreType.DMA,
            pltpu.SemaphoreType.DMA,
        ],
        compiler_params=pltpu.CompilerParams(
            use_tc_tiling_on_sc=False, needs_layout_passes=False),
    )


BN = 2048


def _tr_body(in_ref, out_ref):
    a = in_ref[...]
    bf = a.astype(jnp.bfloat16)
    w = pltpu.bitcast(bf, jnp.int32)
    out_ref[:, 0:D // 2] = w.T


def _transpose_table(wt):
    n = wt.shape[1]
    return pl.pallas_call(
        _tr_body,
        grid=(pl.cdiv(n, BN),),
        in_specs=[pl.BlockSpec((D, BN), lambda i: (0, i))],
        out_specs=pl.BlockSpec((BN, 128), lambda i: (i, 0)),
        out_shape=jax.ShapeDtypeStruct((n, 128), jnp.int32),
        compiler_params=pltpu.CompilerParams(
            dimension_semantics=("arbitrary",)),
    )(wt)


TPJ = 64


def _fmt_body(in_ref, out_ref):
    b = pltpu.bitcast(in_ref[...], jnp.bfloat16)
    for t in range(TPJ):
        out_ref[0, :, pl.ds(CHUNK * t, CHUNK)] = b[D * t:D * (t + 1), :]


def _format_out(ou2):
    rpb = TPJ * D // 2
    return pl.pallas_call(
        _fmt_body,
        grid=(NJ, NI // CHUNK // TPJ),
        in_specs=[pl.BlockSpec((rpb, 128),
                               lambda j, u: (j * (NI // CHUNK // TPJ) + u, 0))],
        out_specs=pl.BlockSpec((1, D, TPJ * CHUNK), lambda j, u: (j, 0, u)),
        out_shape=jax.ShapeDtypeStruct((NJ, D, NI), jnp.bfloat16),
        compiler_params=pltpu.CompilerParams(
            dimension_semantics=("parallel", "parallel")),
    )(ou2)


def kernel(x, weight):
    n_total = x.size
    n_chunk = n_total // (NW * CHUNK)
    xw = x.T.reshape(NW, n_chunk, CHUNK).astype(jnp.int32)
    table = _transpose_table(weight.T)
    ou = _build(n_chunk)(xw, table)
    o3 = _format_out(ou.reshape(NJ * (NI // CHUNK) * (D // 2), CHUNK))
    return o3.transpose(2, 0, 1)


# speedup vs baseline: 3.1343x; 1.3068x over previous
import functools

import jax
import jax.numpy as jnp
from jax import lax
from jax.experimental import pallas as pl
from jax.experimental.pallas import tpu as pltpu
from jax.experimental.pallas import tpu_sc as plsc

D = 64
NC, NS = 2, 16
NW = NC * NS
CHUNK = 128
L = 16
NJ = 50
NI = 16384


def _cast_block(rows_ref, stg_ref):
    iota = lax.iota(jnp.int32, L)

    def rr_body(rr, _):
        rk = (rr + iota) & (D // 2 - 1)
        for g in range(CHUNK // L):
            cl = g * L + iota
            w = plsc.load_gather(rows_ref, [cl, rk])
            plsc.store_scatter(stg_ref, [rk, cl], w)
        return 0

    lax.fori_loop(0, D // 2, rr_body, 0, unroll=4)


def _emb_body(n_chunk, x_hbm, w_hbm, out_hbm, idx_v, r0, r1, s0, s1,
              sem_i, g0, g1, w0, w1):
    wid = lax.axis_index("s") * NC + lax.axis_index("c")
    base = wid * n_chunk
    pltpu.async_copy(x_hbm.at[wid], idx_v, sem_i).wait()

    rows = (r0, r1)
    stgs = (s0, s1)
    gs = (g0, g1)
    ws = (w0, w1)

    def fire_gather(c, b):
        pltpu.async_copy(w_hbm.at[idx_v.at[c]], rows[b], gs[b])

    def wait_gather(c, b):
        pltpu.make_async_copy(w_hbm.at[idx_v.at[c]], rows[b], gs[b]).wait()

    def fire_write(c, b):
        blk = base + c
        j = blk // (NI // CHUNK)
        tc = blk % (NI // CHUNK)
        pltpu.async_copy(stgs[b], out_hbm.at[j, tc], ws[b])

    def drain_write(b):
        pltpu.make_async_copy(stgs[b], out_hbm.at[0, 0], ws[b]).wait()

    fire_gather(0, 0)
    fire_gather(1, 1)

    for b in range(2):
        wait_gather(b, b)
        _cast_block(rows[b], stgs[b])
        fire_write(b, b)
        fire_gather(b + 2, b)

    half = n_chunk // 2

    def main_body(k, _):
        c = 2 * k
        for b in range(2):
            wait_gather(c + b, b)
            drain_write(b)
            _cast_block(rows[b], stgs[b])
            fire_write(c + b, b)
            fire_gather(c + b + 2, b)
        return 0

    lax.fori_loop(1, half - 1, main_body, 0)

    for b in range(2):
        c = n_chunk - 2 + b
        wait_gather(c, b)
        drain_write(b)
        _cast_block(rows[b], stgs[b])
        fire_write(c, b)

    drain_write(0)
    drain_write(1)


def _build(n_chunk):
    mesh = plsc.VectorSubcoreMesh(core_axis_name="c", subcore_axis_name="s")
    return pl.kernel(
        functools.partial(_emb_body, n_chunk),
        out_type=jax.ShapeDtypeStruct((NJ, NI // CHUNK, D // 2, CHUNK),
                                      jnp.int32),
        mesh=mesh,
        scratch_types=[
            pltpu.VMEM((n_chunk, CHUNK), jnp.int32),
            pltpu.VMEM((CHUNK, 128), jnp.int32),
            pltpu.VMEM((CHUNK, 128), jnp.int32),
            pltpu.VMEM((D // 2, CHUNK), jnp.int32),
            pltpu.VMEM((D // 2, CHUNK), jnp.int32),
            pltpu.SemaphoreType.DMA,
            pltpu.SemaphoreType.DMA,
            pltpu.SemaphoreType.DMA,
            pltpu.SemaphoreType.DMA,
            pltpu.SemaphoreType.DMA,
        ],
        compiler_params=pltpu.CompilerParams(
            use_tc_tiling_on_sc=False, needs_layout_passes=False),
    )


BN = 8192


def _tr_body(in_ref, out_ref):
    a = in_ref[...]
    bf = a.astype(jnp.bfloat16)
    w = pltpu.bitcast(bf, jnp.int32)
    out_ref[:, 0:D // 2] = w.T


def _transpose_table(wt):
    n = wt.shape[1]
    return pl.pallas_call(
        _tr_body,
        grid=(pl.cdiv(n, BN),),
        in_specs=[pl.BlockSpec((D, BN), lambda i: (0, i))],
        out_specs=pl.BlockSpec((BN, 128), lambda i: (i, 0)),
        out_shape=jax.ShapeDtypeStruct((n, 128), jnp.int32),
        compiler_params=pltpu.CompilerParams(
            dimension_semantics=("arbitrary",)),
    )(wt)


TPJ = 64


def _fmt_body(in_ref, out_ref):
    b = pltpu.bitcast(in_ref[...], jnp.bfloat16)
    for t in range(TPJ):
        out_ref[0, :, pl.ds(CHUNK * t, CHUNK)] = b[D * t:D * (t + 1), :]


def _format_out(ou2):
    rpb = TPJ * D // 2
    return pl.pallas_call(
        _fmt_body,
        grid=(NJ, NI // CHUNK // TPJ),
        in_specs=[pl.BlockSpec((rpb, 128),
                               lambda j, u: (j * (NI // CHUNK // TPJ) + u, 0))],
        out_specs=pl.BlockSpec((1, D, TPJ * CHUNK), lambda j, u: (j, 0, u)),
        out_shape=jax.ShapeDtypeStruct((NJ, D, NI), jnp.bfloat16),
        compiler_params=pltpu.CompilerParams(
            dimension_semantics=("parallel", "parallel")),
    )(ou2)


def kernel(x, weight):
    n_total = x.size
    n_chunk = n_total // (NW * CHUNK)
    xw = x.T.reshape(NW, n_chunk, CHUNK).astype(jnp.int32)
    table = _transpose_table(weight.T)
    ou = _build(n_chunk)(xw, table)
    o3 = _format_out(ou.reshape(NJ * (NI // CHUNK) * (D // 2), CHUNK))
    return o3.transpose(2, 0, 1)


# speedup vs baseline: 3.2244x; 1.0287x over previous
import functools

import jax
import jax.numpy as jnp
from jax import lax
from jax.experimental import pallas as pl
from jax.experimental.pallas import tpu as pltpu
from jax.experimental.pallas import tpu_sc as plsc

D = 64
NC, NS = 2, 16
NW = NC * NS
CHUNK = 128
L = 16
NJ = 50
NI = 16384


def _cast_block(rows_ref, stg_ref):
    iota = lax.iota(jnp.int32, L)

    def rr_body(rr, _):
        rk = (rr + iota) & (D // 2 - 1)
        for g in range(CHUNK // L):
            cl = g * L + iota
            w = plsc.load_gather(rows_ref, [cl, rk])
            plsc.store_scatter(stg_ref, [rk, cl], w)
        return 0

    lax.fori_loop(0, D // 2, rr_body, 0, unroll=4)


def _emb_body(n_chunk, x_hbm, w_hbm, out_hbm, idx_v, r0, r1, s0, s1,
              sem_i, g0, g1, w0, w1):
    wid = lax.axis_index("s") * NC + lax.axis_index("c")
    base = wid * n_chunk
    pltpu.async_copy(x_hbm.at[wid], idx_v, sem_i).wait()

    rows = (r0, r1)
    stgs = (s0, s1)
    gs = (g0, g1)
    ws = (w0, w1)

    def fire_gather(c, b):
        pltpu.async_copy(w_hbm.at[idx_v.at[c]], rows[b], gs[b])

    def wait_gather(c, b):
        pltpu.make_async_copy(w_hbm.at[idx_v.at[c]], rows[b], gs[b]).wait()

    def fire_write(c, b):
        blk = base + c
        j = blk // (NI // CHUNK)
        tc = blk % (NI // CHUNK)
        pltpu.async_copy(stgs[b], out_hbm.at[j, tc], ws[b])

    def drain_write(b):
        pltpu.make_async_copy(stgs[b], out_hbm.at[0, 0], ws[b]).wait()

    fire_gather(0, 0)
    fire_gather(1, 1)

    for b in range(2):
        wait_gather(b, b)
        _cast_block(rows[b], stgs[b])
        fire_write(b, b)
        fire_gather(b + 2, b)

    half = n_chunk // 2

    def main_body(k, _):
        c = 2 * k
        for b in range(2):
            wait_gather(c + b, b)
            drain_write(b)
            _cast_block(rows[b], stgs[b])
            fire_write(c + b, b)
            fire_gather(c + b + 2, b)
        return 0

    lax.fori_loop(1, half - 1, main_body, 0)

    for b in range(2):
        c = n_chunk - 2 + b
        wait_gather(c, b)
        drain_write(b)
        _cast_block(rows[b], stgs[b])
        fire_write(c, b)

    drain_write(0)
    drain_write(1)


def _build(n_chunk):
    mesh = plsc.VectorSubcoreMesh(core_axis_name="c", subcore_axis_name="s")
    return pl.kernel(
        functools.partial(_emb_body, n_chunk),
        out_type=jax.ShapeDtypeStruct((NJ, NI // CHUNK, D // 2, CHUNK),
                                      jnp.int32),
        mesh=mesh,
        scratch_types=[
            pltpu.VMEM((n_chunk, CHUNK), jnp.int32),
            pltpu.VMEM((CHUNK, 128), jnp.int32),
            pltpu.VMEM((CHUNK, 128), jnp.int32),
            pltpu.VMEM((D // 2, CHUNK), jnp.int32),
            pltpu.VMEM((D // 2, CHUNK), jnp.int32),
            pltpu.SemaphoreType.DMA,
            pltpu.SemaphoreType.DMA,
            pltpu.SemaphoreType.DMA,
            pltpu.SemaphoreType.DMA,
            pltpu.SemaphoreType.DMA,
        ],
        compiler_params=pltpu.CompilerParams(
            use_tc_tiling_on_sc=False, needs_layout_passes=False),
    )


BN = 16384


def _tr_body(in_ref, out_ref):
    a = in_ref[...]
    bf = a.astype(jnp.bfloat16)
    w = pltpu.bitcast(bf, jnp.int32)
    out_ref[:, 0:D // 2] = w.T


def _transpose_table(wt):
    n = wt.shape[1]
    return pl.pallas_call(
        _tr_body,
        grid=(pl.cdiv(n, BN),),
        in_specs=[pl.BlockSpec((D, BN), lambda i: (0, i))],
        out_specs=pl.BlockSpec((BN, 128), lambda i: (i, 0)),
        out_shape=jax.ShapeDtypeStruct((n, 128), jnp.int32),
        compiler_params=pltpu.CompilerParams(
            dimension_semantics=("arbitrary",)),
    )(wt)


TPJ = 64


def _fmt_body(in_ref, out_ref):
    b = pltpu.bitcast(in_ref[...], jnp.bfloat16)
    for t in range(TPJ):
        out_ref[0, :, pl.ds(CHUNK * t, CHUNK)] = b[D * t:D * (t + 1), :]


def _format_out(ou2):
    rpb = TPJ * D // 2
    return pl.pallas_call(
        _fmt_body,
        grid=(NJ, NI // CHUNK // TPJ),
        in_specs=[pl.BlockSpec((rpb, 128),
                               lambda j, u: (j * (NI // CHUNK // TPJ) + u, 0))],
        out_specs=pl.BlockSpec((1, D, TPJ * CHUNK), lambda j, u: (j, 0, u)),
        out_shape=jax.ShapeDtypeStruct((NJ, D, NI), jnp.bfloat16),
        compiler_params=pltpu.CompilerParams(
            dimension_semantics=("parallel", "parallel")),
    )(ou2)


def kernel(x, weight):
    n_total = x.size
    n_chunk = n_total // (NW * CHUNK)
    xw = x.T.reshape(NW, n_chunk, CHUNK).astype(jnp.int32)
    table = _transpose_table(weight.T)
    ou = _build(n_chunk)(xw, table)
    o3 = _format_out(ou.reshape(NJ * (NI // CHUNK) * (D // 2), CHUNK))
    return o3.transpose(2, 0, 1)


# speedup vs baseline: 3.4990x; 1.0852x over previous
import functools

import jax
import jax.numpy as jnp
from jax import lax
from jax.experimental import pallas as pl
from jax.experimental.pallas import tpu as pltpu
from jax.experimental.pallas import tpu_sc as plsc

D = 64
NC, NS = 2, 16
NW = NC * NS
CHUNK = 128
L = 16
NJ = 50
NI = 16384


def _cast_block(rows_ref, stg_ref):
    iota = lax.iota(jnp.int32, L)

    def rr_body(rr, _):
        rk = (rr + iota) & (D // 2 - 1)
        for g in range(CHUNK // L):
            cl = g * L + iota
            w = plsc.load_gather(rows_ref, [cl, rk])
            plsc.store_scatter(stg_ref, [rk, cl], w)
        return 0

    lax.fori_loop(0, D // 2, rr_body, 0, unroll=4)


NBUF = 4


def _emb_body(n_chunk, x_hbm, w_hbm, out_hbm, idx_v, r0, r1, r2, r3,
              s0, s1, s2, s3, sem_i, g0, g1, g2, g3, w0, w1, w2, w3):
    wid = lax.axis_index("s") * NC + lax.axis_index("c")
    base = wid * n_chunk
    pltpu.async_copy(x_hbm.at[wid], idx_v, sem_i).wait()

    rows = (r0, r1, r2, r3)
    stgs = (s0, s1, s2, s3)
    gs = (g0, g1, g2, g3)
    ws = (w0, w1, w2, w3)

    def fire_gather(c, b):
        pltpu.async_copy(w_hbm.at[idx_v.at[c]], rows[b], gs[b])

    def wait_gather(c, b):
        pltpu.make_async_copy(w_hbm.at[idx_v.at[c]], rows[b], gs[b]).wait()

    def fire_write(c, b):
        blk = base + c
        j = blk // (NI // CHUNK)
        tc = blk % (NI // CHUNK)
        pltpu.async_copy(stgs[b], out_hbm.at[j, tc], ws[b])

    def drain_write(b):
        pltpu.make_async_copy(stgs[b], out_hbm.at[0, 0], ws[b]).wait()

    for b in range(NBUF):
        fire_gather(b, b)

    for b in range(NBUF):
        wait_gather(b, b)
        _cast_block(rows[b], stgs[b])
        fire_write(b, b)
        fire_gather(b + NBUF, b)

    def main_body(k, _):
        c = NBUF * k
        for b in range(NBUF):
            wait_gather(c + b, b)
            drain_write(b)
            _cast_block(rows[b], stgs[b])
            fire_write(c + b, b)
            fire_gather(c + b + NBUF, b)
        return 0

    lax.fori_loop(1, n_chunk // NBUF - 1, main_body, 0)

    for b in range(NBUF):
        c = n_chunk - NBUF + b
        wait_gather(c, b)
        drain_write(b)
        _cast_block(rows[b], stgs[b])
        fire_write(c, b)

    for b in range(NBUF):
        drain_write(b)


def _build(n_chunk):
    mesh = plsc.VectorSubcoreMesh(core_axis_name="c", subcore_axis_name="s")
    return pl.kernel(
        functools.partial(_emb_body, n_chunk),
        out_type=jax.ShapeDtypeStruct((NJ, NI // CHUNK, D // 2, CHUNK),
                                      jnp.int32),
        mesh=mesh,
        scratch_types=[
            pltpu.VMEM((n_chunk, CHUNK), jnp.int32),
            pltpu.VMEM((CHUNK, 128), jnp.int32),
            pltpu.VMEM((CHUNK, 128), jnp.int32),
            pltpu.VMEM((CHUNK, 128), jnp.int32),
            pltpu.VMEM((CHUNK, 128), jnp.int32),
            pltpu.VMEM((D // 2, CHUNK), jnp.int32),
            pltpu.VMEM((D // 2, CHUNK), jnp.int32),
            pltpu.VMEM((D // 2, CHUNK), jnp.int32),
            pltpu.VMEM((D // 2, CHUNK), jnp.int32),
            pltpu.SemaphoreType.DMA,
            pltpu.SemaphoreType.DMA,
            pltpu.SemaphoreType.DMA,
            pltpu.SemaphoreType.DMA,
            pltpu.SemaphoreType.DMA,
            pltpu.SemaphoreType.DMA,
            pltpu.SemaphoreType.DMA,
            pltpu.SemaphoreType.DMA,
            pltpu.SemaphoreType.DMA,
        ],
        compiler_params=pltpu.CompilerParams(
            use_tc_tiling_on_sc=False, needs_layout_passes=False),
    )


BN = 16384


def _tr_body(in_ref, out_ref):
    a = in_ref[...]
    bf = a.astype(jnp.bfloat16)
    w = pltpu.bitcast(bf, jnp.int32)
    out_ref[:, 0:D // 2] = w.T


def _transpose_table(wt):
    n = wt.shape[1]
    return pl.pallas_call(
        _tr_body,
        grid=(pl.cdiv(n, BN),),
        in_specs=[pl.BlockSpec((D, BN), lambda i: (0, i))],
        out_specs=pl.BlockSpec((BN, 128), lambda i: (i, 0)),
        out_shape=jax.ShapeDtypeStruct((n, 128), jnp.int32),
        compiler_params=pltpu.CompilerParams(
            dimension_semantics=("arbitrary",)),
    )(wt)


TPJ = 64


def _fmt_body(in_ref, out_ref):
    b = pltpu.bitcast(in_ref[...], jnp.bfloat16)
    for t in range(TPJ):
        out_ref[0, :, pl.ds(CHUNK * t, CHUNK)] = b[D * t:D * (t + 1), :]


def _format_out(ou2):
    rpb = TPJ * D // 2
    return pl.pallas_call(
        _fmt_body,
        grid=(NJ, NI // CHUNK // TPJ),
        in_specs=[pl.BlockSpec((rpb, 128),
                               lambda j, u: (j * (NI // CHUNK // TPJ) + u, 0))],
        out_specs=pl.BlockSpec((1, D, TPJ * CHUNK), lambda j, u: (j, 0, u)),
        out_shape=jax.ShapeDtypeStruct((NJ, D, NI), jnp.bfloat16),
        compiler_params=pltpu.CompilerParams(
            dimension_semantics=("parallel", "parallel")),
    )(ou2)


def kernel(x, weight):
    n_total = x.size
    n_chunk = n_total // (NW * CHUNK)
    xw = x.T.reshape(NW, n_chunk, CHUNK).astype(jnp.int32)
    table = _transpose_table(weight.T)
    ou = _build(n_chunk)(xw, table)
    o3 = _format_out(ou.reshape(NJ * (NI // CHUNK) * (D // 2), CHUNK))
    return o3.transpose(2, 0, 1)


# speedup vs baseline: 3.5411x; 1.0120x over previous
import functools

import jax
import jax.numpy as jnp
from jax import lax
from jax.experimental import pallas as pl
from jax.experimental.pallas import tpu as pltpu
from jax.experimental.pallas import tpu_sc as plsc

D = 64
NC, NS = 2, 16
NW = NC * NS
CHUNK = 128
L = 16
NJ = 50
NI = 16384


def _cast_block(rows_ref, stg_ref):
    iota = lax.iota(jnp.int32, L)

    def rr_body(rr, _):
        rk = (rr + iota) & (D // 2 - 1)
        for g in range(CHUNK // L):
            cl = g * L + iota
            w = plsc.load_gather(rows_ref, [cl, rk])
            plsc.store_scatter(stg_ref, [rk, cl], w)
        return 0

    lax.fori_loop(0, D // 2, rr_body, 0, unroll=4)


NBUF = 4


def _emb_body(n_chunk, x_hbm, w_hbm, out_hbm, idx_v, r0, r1, r2, r3,
              s0, s1, s2, s3, sem_i, g0, g1, g2, g3, w0, w1, w2, w3):
    wid = lax.axis_index("s") * NC + lax.axis_index("c")
    base = wid * n_chunk
    pltpu.async_copy(x_hbm.at[wid], idx_v, sem_i).wait()

    rows = (r0, r1, r2, r3)
    stgs = (s0, s1, s2, s3)
    gs = (g0, g1, g2, g3)
    ws = (w0, w1, w2, w3)

    def fire_gather(c, b):
        pltpu.async_copy(w_hbm.at[idx_v.at[c]], rows[b], gs[b])

    def wait_gather(c, b):
        pltpu.make_async_copy(w_hbm.at[idx_v.at[c]], rows[b], gs[b]).wait()

    def fire_write(c, b):
        blk = base + c
        j = blk // (NI // CHUNK)
        tc = blk % (NI // CHUNK)
        pltpu.async_copy(stgs[b], out_hbm.at[j, tc], ws[b])

    def drain_write(b):
        pltpu.make_async_copy(stgs[b], out_hbm.at[0, 0], ws[b]).wait()

    for b in range(NBUF):
        fire_gather(b, b)

    for b in range(NBUF):
        wait_gather(b, b)
        _cast_block(rows[b], stgs[b])
        fire_write(b, b)
        fire_gather(b + NBUF, b)

    def main_body(k, _):
        c = NBUF * k
        for b in range(NBUF):
            wait_gather(c + b, b)
            drain_write(b)
            _cast_block(rows[b], stgs[b])
            fire_write(c + b, b)
            fire_gather(c + b + NBUF, b)
        return 0

    lax.fori_loop(1, n_chunk // NBUF - 1, main_body, 0)

    for b in range(NBUF):
        c = n_chunk - NBUF + b
        wait_gather(c, b)
        drain_write(b)
        _cast_block(rows[b], stgs[b])
        fire_write(c, b)

    for b in range(NBUF):
        drain_write(b)


def _build(n_chunk):
    mesh = plsc.VectorSubcoreMesh(core_axis_name="c", subcore_axis_name="s")
    return pl.kernel(
        functools.partial(_emb_body, n_chunk),
        out_type=jax.ShapeDtypeStruct((NJ, NI // CHUNK, D // 2, CHUNK),
                                      jnp.int32),
        mesh=mesh,
        scratch_types=[
            pltpu.VMEM((n_chunk, CHUNK), jnp.int32),
            pltpu.VMEM((CHUNK, 128), jnp.int32),
            pltpu.VMEM((CHUNK, 128), jnp.int32),
            pltpu.VMEM((CHUNK, 128), jnp.int32),
            pltpu.VMEM((CHUNK, 128), jnp.int32),
            pltpu.VMEM((D // 2, CHUNK), jnp.int32),
            pltpu.VMEM((D // 2, CHUNK), jnp.int32),
            pltpu.VMEM((D // 2, CHUNK), jnp.int32),
            pltpu.VMEM((D // 2, CHUNK), jnp.int32),
            pltpu.SemaphoreType.DMA,
            pltpu.SemaphoreType.DMA,
            pltpu.SemaphoreType.DMA,
            pltpu.SemaphoreType.DMA,
            pltpu.SemaphoreType.DMA,
            pltpu.SemaphoreType.DMA,
            pltpu.SemaphoreType.DMA,
            pltpu.SemaphoreType.DMA,
            pltpu.SemaphoreType.DMA,
        ],
        compiler_params=pltpu.CompilerParams(
            use_tc_tiling_on_sc=False, needs_layout_passes=False),
    )


BN = 32768


def _tr_body(in_ref, out_ref):
    a = in_ref[...]
    bf = a.astype(jnp.bfloat16)
    w = pltpu.bitcast(bf, jnp.int32)
    out_ref[:, 0:D // 2] = w.T


def _transpose_table(wt):
    n = wt.shape[1]
    return pl.pallas_call(
        _tr_body,
        grid=(pl.cdiv(n, BN),),
        in_specs=[pl.BlockSpec((D, BN), lambda i: (0, i))],
        out_specs=pl.BlockSpec((BN, 128), lambda i: (i, 0)),
        out_shape=jax.ShapeDtypeStruct((n, 128), jnp.int32),
        compiler_params=pltpu.CompilerParams(
            dimension_semantics=("arbitrary",)),
    )(wt)


TPJ = 64


def _fmt_body(in_ref, out_ref):
    b = pltpu.bitcast(in_ref[...], jnp.bfloat16)
    for t in range(TPJ):
        out_ref[0, :, pl.ds(CHUNK * t, CHUNK)] = b[D * t:D * (t + 1), :]


def _format_out(ou2):
    rpb = TPJ * D // 2
    return pl.pallas_call(
        _fmt_body,
        grid=(NJ, NI // CHUNK // TPJ),
        in_specs=[pl.BlockSpec((rpb, 128),
                               lambda j, u: (j * (NI // CHUNK // TPJ) + u, 0))],
        out_specs=pl.BlockSpec((1, D, TPJ * CHUNK), lambda j, u: (j, 0, u)),
        out_shape=jax.ShapeDtypeStruct((NJ, D, NI), jnp.bfloat16),
        compiler_params=pltpu.CompilerParams(
            dimension_semantics=("parallel", "parallel")),
    )(ou2)


def kernel(x, weight):
    n_total = x.size
    n_chunk = n_total // (NW * CHUNK)
    xw = x.T.reshape(NW, n_chunk, CHUNK).astype(jnp.int32)
    table = _transpose_table(weight.T)
    ou = _build(n_chunk)(xw, table)
    o3 = _format_out(ou.reshape(NJ * (NI // CHUNK) * (D // 2), CHUNK))
    return o3.transpose(2, 0, 1)
